# baseline (speedup 1.0000x reference)
import functools

import jax
import jax.numpy as jnp
from jax import lax
from jax.experimental import pallas as pl
from jax.experimental.pallas import tpu as pltpu
from jax.experimental.pallas import tpu_sc as plsc

B = 16384
D = 128
V = 100
NC = 2
NS = 16
NW = NC * NS
B_PER_W = B // NW
CHUNK = 128
N_CHUNKS = B_PER_W // CHUNK
L = 16
BLOCKS_PER_CHUNK = CHUNK // L


def _mod_v(x):
    hi = lax.shift_right_logical(x, 10)
    lo = lax.bitwise_and(x, 1023)
    y = hi * 24 + lo
    q = lax.shift_right_logical(y * 20972, 21)
    return y - q * V


def _sc_body(uid_hbm, table_hbm, out_hbm, idx_v, table_v, rows_v, sem_w):
    wid = lax.axis_index("s") * NC + lax.axis_index("c")
    base = wid * B_PER_W

    pltpu.sync_copy(uid_hbm.at[wid], idx_v)
    pltpu.sync_copy(table_hbm, table_v)

    for i in range(B_PER_W // L):
        sl = pl.ds(i * L, L)
        idx_v[sl] = _mod_v(idx_v[sl])

    lane = lax.iota(jnp.int32, L)

    writes = []
    for j in range(N_CHUNKS):
        half = j % 2
        if j >= 2:
            writes[j - 2].wait()

        def block_body(b, _):
            rows = idx_v[pl.ds(j * CHUNK + b * L, L)]
            orows = half * CHUNK + b * L + lane
            for c in range(D):
                cvec = jnp.full((L,), c, jnp.int32)
                val = plsc.load_gather(table_v, [rows, cvec])
                plsc.store_scatter(rows_v, [orows, cvec], val)
            return 0

        lax.fori_loop(0, BLOCKS_PER_CHUNK, block_body, 0)

        writes.append(
            pltpu.async_copy(
                rows_v.at[pl.ds(half * CHUNK, CHUNK)],
                out_hbm.at[pl.ds(base + j * CHUNK, CHUNK)],
                sem_w,
            )
        )
    for w in writes[-2:]:
        w.wait()


def kernel(user_id, user_embeddings):
    uid = user_id.astype(jnp.int32).reshape(NW, B_PER_W)
    table = user_embeddings.astype(jnp.float32)

    mesh = plsc.VectorSubcoreMesh(core_axis_name="c", subcore_axis_name="s")
    run = pl.kernel(
        _sc_body,
        mesh=mesh,
        compiler_params=pltpu.CompilerParams(needs_layout_passes=False),
        out_type=jax.ShapeDtypeStruct((B, D), jnp.float32),
        scratch_types=[
            pltpu.VMEM((B_PER_W,), jnp.int32),
            pltpu.VMEM((V, D), jnp.float32),
            pltpu.VMEM((2 * CHUNK, D), jnp.float32),
            pltpu.SemaphoreType.DMA,
        ],
    )
    return run(uid, table)


# speedup vs baseline: 2.3482x; 2.3482x over previous
import functools

import jax
import jax.numpy as jnp
from jax import lax
from jax.experimental import pallas as pl
from jax.experimental.pallas import tpu as pltpu
from jax.experimental.pallas import tpu_sc as plsc

B = 16384
D = 128
V = 100
NC = 2
NS = 16
NW = NC * NS
B_PER_W = B // NW
CHUNK = 128
N_CHUNKS = B_PER_W // CHUNK
L = 16
BLOCKS_PER_CHUNK = CHUNK // L


def _mod_v(x):
    hi = lax.shift_right_logical(x, 10)
    lo = lax.bitwise_and(x, 1023)
    y = hi * 24 + lo
    q = lax.shift_right_logical(y * 20972, 21)
    return y - q * V


def _sc_body(uid_hbm, table_hbm, out_hbm, idx_v, table_v, rows_v, sem_w):
    wid = lax.axis_index("s") * NC + lax.axis_index("c")
    base = wid * B_PER_W

    pltpu.sync_copy(uid_hbm.at[wid], idx_v)
    pltpu.sync_copy(table_hbm, table_v)

    for i in range(B_PER_W // L):
        sl = pl.ds(i * L, L)
        idx_v[sl] = _mod_v(idx_v[sl]) * D

    lane = lax.iota(jnp.int32, L)

    writes = []
    for j in range(N_CHUNKS):
        half = j % 2
        if j >= 2:
            writes[j - 2].wait()

        def block_body(b, _):
            rb = idx_v[pl.ds(j * CHUNK + b * L, L)]
            ob = (half * CHUNK + b * L + lane) * D
            cvec = lane
            for s in range(D):
                gi = rb + cvec
                val = plsc.load_gather(table_v, [gi])
                plsc.store_scatter(rows_v, [ob + cvec], val)
                cvec = lax.bitwise_and(cvec + 1, D - 1)
            return 0

        lax.fori_loop(0, BLOCKS_PER_CHUNK, block_body, 0)

        writes.append(
            pltpu.async_copy(
                rows_v.at[pl.ds(half * CHUNK * D, CHUNK * D)],
                out_hbm.at[pl.ds((base + j * CHUNK) * D, CHUNK * D)],
                sem_w,
            )
        )
    for w in writes[-2:]:
        w.wait()


def kernel(user_id, user_embeddings):
    uid = user_id.astype(jnp.int32).reshape(NW, B_PER_W)
    table = user_embeddings.astype(jnp.float32).reshape(V * D)

    mesh = plsc.VectorSubcoreMesh(core_axis_name="c", subcore_axis_name="s")
    run = pl.kernel(
        _sc_body,
        mesh=mesh,
        compiler_params=pltpu.CompilerParams(needs_layout_passes=False),
        out_type=jax.ShapeDtypeStruct((B * D,), jnp.float32),
        scratch_types=[
            pltpu.VMEM((B_PER_W,), jnp.int32),
            pltpu.VMEM((V * D,), jnp.float32),
            pltpu.VMEM((2 * CHUNK * D,), jnp.float32),
            pltpu.SemaphoreType.DMA,
        ],
    )
    return run(uid, table).reshape(B, D)


# speedup vs baseline: 2.4632x; 1.0490x over previous
import functools

import jax
import jax.numpy as jnp
from jax import lax
from jax.experimental import pallas as pl
from jax.experimental.pallas import tpu as pltpu
from jax.experimental.pallas import tpu_sc as plsc

B = 16384
D = 128
V = 100
NC = 2
NS = 16
NW = NC * NS
B_PER_W = B // NW
CHUNK = 128
N_CHUNKS = B_PER_W // CHUNK
L = 16
BLOCKS_PER_CHUNK = CHUNK // L


def _mod_v(x):
    hi = lax.shift_right_logical(x, 10)
    lo = lax.bitwise_and(x, 1023)
    y = hi * 24 + lo
    q = lax.shift_right_logical(y * 20972, 21)
    return y - q * V


def _sc_body(uid_hbm, table_hbm, out_hbm, idx_v, table_v, rows_v, sem_w):
    wid = lax.axis_index("s") * NC + lax.axis_index("c")
    base = wid * B_PER_W

    pltpu.sync_copy(uid_hbm.at[wid], idx_v)
    pltpu.sync_copy(table_hbm, table_v)

    for i in range(B_PER_W // L):
        sl = pl.ds(i * L, L)
        idx_v[sl] = _mod_v(idx_v[sl]) * D

    lane = lax.iota(jnp.int32, L)

    writes = []
    for j in range(N_CHUNKS):
        half = j % 2
        if j >= 2:
            writes[j - 2].wait()

        NI = 4

        def block_body(bp, _):
            b0 = bp * NI
            rbs = [idx_v[pl.ds(j * CHUNK + (b0 + k) * L, L)] for k in range(NI)]
            obs = [(half * CHUNK + (b0 + k) * L + lane) * D for k in range(NI)]
            cvec = lane
            for s in range(D):
                gis = [rbs[k] + cvec for k in range(NI)]
                vals = [plsc.load_gather(table_v, [gis[k]]) for k in range(NI)]
                for k in range(NI):
                    plsc.store_scatter(rows_v, [obs[k] + cvec], vals[k])
                cvec = lax.bitwise_and(cvec + 1, D - 1)
            return 0

        lax.fori_loop(0, BLOCKS_PER_CHUNK // NI, block_body, 0)

        writes.append(
            pltpu.async_copy(
                rows_v.at[pl.ds(half * CHUNK * D, CHUNK * D)],
                out_hbm.at[pl.ds((base + j * CHUNK) * D, CHUNK * D)],
                sem_w,
            )
        )
    for w in writes[-2:]:
        w.wait()


def kernel(user_id, user_embeddings):
    uid = user_id.astype(jnp.int32).reshape(NW, B_PER_W)
    table = user_embeddings.astype(jnp.float32).reshape(V * D)

    mesh = plsc.VectorSubcoreMesh(core_axis_name="c", subcore_axis_name="s")
    run = pl.kernel(
        _sc_body,
        mesh=mesh,
        compiler_params=pltpu.CompilerParams(needs_layout_passes=False),
        out_type=jax.ShapeDtypeStruct((B * D,), jnp.float32),
        scratch_types=[
            pltpu.VMEM((B_PER_W,), jnp.int32),
            pltpu.VMEM((V * D,), jnp.float32),
            pltpu.VMEM((2 * CHUNK * D,), jnp.float32),
            pltpu.SemaphoreType.DMA,
        ],
    )
    return run(uid, table).reshape(B, D)


# speedup vs baseline: 2.6343x; 1.0695x over previous
import functools

import jax
import jax.numpy as jnp
from jax import lax
from jax.experimental import pallas as pl
from jax.experimental.pallas import tpu as pltpu
from jax.experimental.pallas import tpu_sc as plsc

B = 16384
D = 128
V = 100
NC = 2
NS = 16
NW = NC * NS
B_PER_W = B // NW
CHUNK = 128
N_CHUNKS = B_PER_W // CHUNK
L = 16
BLOCKS_PER_CHUNK = CHUNK // L


def _mod_v(x):
    hi = lax.shift_right_logical(x, 10)
    lo = lax.bitwise_and(x, 1023)
    y = hi * 24 + lo
    q = lax.shift_right_logical(y * 20972, 21)
    return y - q * V


def _sc_body(uid_hbm, table_hbm, out_hbm, idx_v, table_v, rows_v, sem_w):
    wid = lax.axis_index("s") * NC + lax.axis_index("c")
    base = wid * B_PER_W

    pltpu.sync_copy(uid_hbm.at[pl.ds(base, B_PER_W)], idx_v)
    pltpu.sync_copy(table_hbm, table_v)

    for i in range(B_PER_W // L):
        sl = pl.ds(i * L, L)
        idx_v[sl] = _mod_v(idx_v[sl]) * D

    lane = lax.iota(jnp.int32, L)

    writes = []
    for j in range(N_CHUNKS):
        half = j % 2
        if j >= 2:
            writes[j - 2].wait()

        NI = 4
        SS = 32

        def block_body(bp, _):
            b0 = bp * NI
            rbs = [idx_v[pl.ds(j * CHUNK + (b0 + k) * L, L)] for k in range(NI)]
            obs = [(half * CHUNK + (b0 + k) * L + lane) * D for k in range(NI)]

            def step_body(so, _):
                cvec = lax.bitwise_and(lane + so * SS, D - 1)
                for s in range(SS):
                    gis = [rbs[k] + cvec for k in range(NI)]
                    vals = [plsc.load_gather(table_v, [gis[k]]) for k in range(NI)]
                    for k in range(NI):
                        plsc.store_scatter(rows_v, [obs[k] + cvec], vals[k])
                    cvec = lax.bitwise_and(cvec + 1, D - 1)
                return 0

            lax.fori_loop(0, D // SS, step_body, 0)
            return 0

        lax.fori_loop(0, BLOCKS_PER_CHUNK // NI, block_body, 0)

        writes.append(
            pltpu.async_copy(
                rows_v.at[pl.ds(half * CHUNK * D, CHUNK * D)],
                out_hbm.at[pl.ds((base + j * CHUNK) * D, CHUNK * D)],
                sem_w,
            )
        )
    for w in writes[-2:]:
        w.wait()


def kernel(user_id, user_embeddings):
    uid = user_id.astype(jnp.int32)
    table = user_embeddings.astype(jnp.float32).reshape(V * D)

    mesh = plsc.VectorSubcoreMesh(core_axis_name="c", subcore_axis_name="s")
    run = pl.kernel(
        _sc_body,
        mesh=mesh,
        compiler_params=pltpu.CompilerParams(needs_layout_passes=False),
        out_type=jax.ShapeDtypeStruct((B * D,), jnp.float32),
        scratch_types=[
            pltpu.VMEM((B_PER_W,), jnp.int32),
            pltpu.VMEM((V * D,), jnp.float32),
            pltpu.VMEM((2 * CHUNK * D,), jnp.float32),
            pltpu.SemaphoreType.DMA,
        ],
    )
    return run(uid, table).reshape(B, D)


# speedup vs baseline: 2.6723x; 1.0144x over previous
import functools

import jax
import jax.numpy as jnp
from jax import lax
from jax.experimental import pallas as pl
from jax.experimental.pallas import tpu as pltpu
from jax.experimental.pallas import tpu_sc as plsc

B = 16384
D = 128
V = 100
NC = 2
NS = 16
NW = NC * NS
B_PER_W = B // NW
CHUNK = 128
N_CHUNKS = B_PER_W // CHUNK
L = 16
BLOCKS_PER_CHUNK = CHUNK // L


def _mod_v(x):
    hi = lax.shift_right_logical(x, 10)
    lo = lax.bitwise_and(x, 1023)
    y = hi * 24 + lo
    q = lax.shift_right_logical(y * 20972, 21)
    return y - q * V


def _sc_body(uid_hbm, table_hbm, out_hbm, idx_v, table_v, rows_v, sem_w, sem_s):
    wid = lax.axis_index("s") * NC + lax.axis_index("c")
    base = wid * B_PER_W

    idx_cp = pltpu.async_copy(uid_hbm.at[pl.ds(base, B_PER_W)], idx_v, sem_s)
    tab_cp = pltpu.async_copy(table_hbm, table_v, sem_s)
    idx_cp.wait()

    for i in range(B_PER_W // L):
        sl = pl.ds(i * L, L)
        idx_v[sl] = _mod_v(idx_v[sl]) * D

    tab_cp.wait()

    lane = lax.iota(jnp.int32, L)

    writes = []
    for j in range(N_CHUNKS):
        half = j % 2
        if j >= 2:
            writes[j - 2].wait()

        NI = 4
        SS = 32

        def block_body(bp, _):
            b0 = bp * NI
            rbs = [idx_v[pl.ds(j * CHUNK + (b0 + k) * L, L)] for k in range(NI)]
            obs = [(half * CHUNK + (b0 + k) * L + lane) * D for k in range(NI)]

            def step_body(so, _):
                cvec = lax.bitwise_and(lane + so * SS, D - 1)
                for s in range(SS):
                    gis = [rbs[k] + cvec for k in range(NI)]
                    vals = [plsc.load_gather(table_v, [gis[k]]) for k in range(NI)]
                    for k in range(NI):
                        plsc.store_scatter(rows_v, [obs[k] + cvec], vals[k])
                    cvec = lax.bitwise_and(cvec + 1, D - 1)
                return 0

            lax.fori_loop(0, D // SS, step_body, 0, unroll=1)
            return 0

        lax.fori_loop(0, BLOCKS_PER_CHUNK // NI, block_body, 0, unroll=1)

        writes.append(
            pltpu.async_copy(
                rows_v.at[pl.ds(half * CHUNK * D, CHUNK * D)],
                out_hbm.at[pl.ds((base + j * CHUNK) * D, CHUNK * D)],
                sem_w,
            )
        )
    for w in writes[-2:]:
        w.wait()


def kernel(user_id, user_embeddings):
    uid = user_id.astype(jnp.int32)
    table = user_embeddings.astype(jnp.float32).reshape(V * D)

    mesh = plsc.VectorSubcoreMesh(core_axis_name="c", subcore_axis_name="s")
    run = pl.kernel(
        _sc_body,
        mesh=mesh,
        compiler_params=pltpu.CompilerParams(needs_layout_passes=False),
        out_type=jax.ShapeDtypeStruct((B * D,), jnp.float32),
        scratch_types=[
            pltpu.VMEM((B_PER_W,), jnp.int32),
            pltpu.VMEM((V * D,), jnp.float32),
            pltpu.VMEM((2 * CHUNK * D,), jnp.float32),
            pltpu.SemaphoreType.DMA,
            pltpu.SemaphoreType.DMA,
        ],
    )
    return run(uid, table).reshape(B, D)


# speedup vs baseline: 3.0331x; 1.1350x over previous
import functools

import jax
import jax.numpy as jnp
from jax import lax
from jax.experimental import pallas as pl
from jax.experimental.pallas import tpu as pltpu
from jax.experimental.pallas import tpu_sc as plsc

B = 16384
D = 128
V = 100
NC = 2
NS = 16
NW = NC * NS
B_PER_W = B // NW
CHUNK = 128
N_CHUNKS = B_PER_W // CHUNK
L = 16
BLOCKS_PER_CHUNK = CHUNK // L


def _mod_v(x):
    hi = lax.shift_right_logical(x, 10)
    lo = lax.bitwise_and(x, 1023)
    y = hi * 24 + lo
    q = lax.shift_right_logical(y * 20972, 21)
    return y - q * V


def _sc_body(uid_hbm, table_hbm, out_hbm, idx_v, table_v, rows_v, sem_w, sem_s):
    wid = lax.axis_index("s") * NC + lax.axis_index("c")
    base = wid * B_PER_W

    idx_cp = pltpu.async_copy(uid_hbm.at[pl.ds(base, B_PER_W)], idx_v, sem_s)
    tab_cp = pltpu.async_copy(table_hbm, table_v, sem_s)
    idx_cp.wait()

    for i in range(B_PER_W // L):
        sl = pl.ds(i * L, L)
        idx_v[sl] = _mod_v(idx_v[sl]) * D

    tab_cp.wait()

    lane = lax.iota(jnp.int32, L)

    writes = []
    for j in range(N_CHUNKS):
        half = j % 2
        if j >= 2:
            writes[j - 2].wait()

        NI = 4
        SS = 16

        def block_body(bp, _):
            b0 = bp * NI
            rbs = [idx_v[pl.ds(j * CHUNK + (b0 + k) * L, L)] for k in range(NI)]
            obs = [(half * CHUNK + (b0 + k) * L + lane) * D for k in range(NI)]

            def step_body(so, _):
                cvec = lax.bitwise_and(lane + so * SS, D - 1)
                for s in range(SS):
                    gis = [rbs[k] + cvec for k in range(NI)]
                    vals = [plsc.load_gather(table_v, [gis[k]]) for k in range(NI)]
                    for k in range(NI):
                        plsc.store_scatter(rows_v, [obs[k] + cvec], vals[k])
                    cvec = lax.bitwise_and(cvec + 1, D - 1)
                return 0

            lax.fori_loop(0, D // SS, step_body, 0, unroll=1)
            return 0

        lax.fori_loop(0, BLOCKS_PER_CHUNK // NI, block_body, 0, unroll=1)

        writes.append(
            pltpu.async_copy(
                rows_v.at[pl.ds(half * CHUNK * D, CHUNK * D)],
                out_hbm.at[pl.ds((base + j * CHUNK) * D, CHUNK * D)],
                sem_w,
            )
        )
    for w in writes[-2:]:
        w.wait()


def kernel(user_id, user_embeddings):
    uid = user_id.astype(jnp.int32)
    table = user_embeddings.astype(jnp.float32).reshape(V * D)

    mesh = plsc.VectorSubcoreMesh(core_axis_name="c", subcore_axis_name="s")
    run = pl.kernel(
        _sc_body,
        mesh=mesh,
        compiler_params=pltpu.CompilerParams(needs_layout_passes=False),
        out_type=jax.ShapeDtypeStruct((B * D,), jnp.float32),
        scratch_types=[
            pltpu.VMEM((B_PER_W,), jnp.int32),
            pltpu.VMEM((V * D,), jnp.float32),
            pltpu.VMEM((2 * CHUNK * D,), jnp.float32),
            pltpu.SemaphoreType.DMA,
            pltpu.SemaphoreType.DMA,
        ],
    )
    return run(uid, table).reshape(B, D)


# speedup vs baseline: 3.2482x; 1.0709x over previous
import functools

import jax
import jax.numpy as jnp
from jax import lax
from jax.experimental import pallas as pl
from jax.experimental.pallas import tpu as pltpu
from jax.experimental.pallas import tpu_sc as plsc

B = 16384
D = 128
V = 100
NC = 2
NS = 16
NW = NC * NS
B_PER_W = B // NW
CHUNK = 128
N_CHUNKS = B_PER_W // CHUNK
L = 16
BLOCKS_PER_CHUNK = CHUNK // L


def _mod_v(x):
    hi = lax.shift_right_logical(x, 10)
    lo = lax.bitwise_and(x, 1023)
    y = hi * 24 + lo
    q = lax.shift_right_logical(y * 20972, 21)
    return y - q * V


def _sc_body(uid_hbm, table_hbm, out_hbm, idx_v, table_v, rows_v, sem_w, sem_s):
    wid = lax.axis_index("s") * NC + lax.axis_index("c")
    base = wid * B_PER_W

    idx_cp = pltpu.async_copy(uid_hbm.at[pl.ds(base, B_PER_W)], idx_v, sem_s)
    tab_cp = pltpu.async_copy(table_hbm, table_v, sem_s)
    idx_cp.wait()

    for i in range(B_PER_W // L):
        sl = pl.ds(i * L, L)
        idx_v[sl] = _mod_v(idx_v[sl]) * D

    tab_cp.wait()

    lane = lax.iota(jnp.int32, L)

    writes = []
    for j in range(N_CHUNKS):
        half = j % 2
        if j >= 2:
            writes[j - 2].wait()

        NI = 4
        SS = 8

        def block_body(bp, _):
            b0 = bp * NI
            rbs = [idx_v[pl.ds(j * CHUNK + (b0 + k) * L, L)] for k in range(NI)]
            obs = [(half * CHUNK + (b0 + k) * L + lane) * D for k in range(NI)]

            def step_body(so, _):
                cvec = lax.bitwise_and(lane + so * SS, D - 1)
                for s in range(SS):
                    gis = [rbs[k] + cvec for k in range(NI)]
                    vals = [plsc.load_gather(table_v, [gis[k]]) for k in range(NI)]
                    for k in range(NI):
                        plsc.store_scatter(rows_v, [obs[k] + cvec], vals[k])
                    cvec = lax.bitwise_and(cvec + 1, D - 1)
                return 0

            lax.fori_loop(0, D // SS, step_body, 0, unroll=1)
            return 0

        lax.fori_loop(0, BLOCKS_PER_CHUNK // NI, block_body, 0, unroll=1)

        writes.append(
            pltpu.async_copy(
                rows_v.at[pl.ds(half * CHUNK * D, CHUNK * D)],
                out_hbm.at[pl.ds((base + j * CHUNK) * D, CHUNK * D)],
                sem_w,
            )
        )
    for w in writes[-2:]:
        w.wait()


def kernel(user_id, user_embeddings):
    uid = user_id.astype(jnp.int32)
    table = user_embeddings.astype(jnp.float32).reshape(V * D)

    mesh = plsc.VectorSubcoreMesh(core_axis_name="c", subcore_axis_name="s")
    run = pl.kernel(
        _sc_body,
        mesh=mesh,
        compiler_params=pltpu.CompilerParams(needs_layout_passes=False),
        out_type=jax.ShapeDtypeStruct((B * D,), jnp.float32),
        scratch_types=[
            pltpu.VMEM((B_PER_W,), jnp.int32),
            pltpu.VMEM((V * D,), jnp.float32),
            pltpu.VMEM((2 * CHUNK * D,), jnp.float32),
            pltpu.SemaphoreType.DMA,
            pltpu.SemaphoreType.DMA,
        ],
    )
    return run(uid, table).reshape(B, D)
